# Initial kernel scaffold; baseline (speedup 1.0000x reference)
import jax
import jax.numpy as jnp
from jax.experimental import pallas as pl


def kernel(x, y, anchors):
    raise NotImplementedError("write your pallas kernel here")


# speedup vs baseline: 6.5033x; 6.5033x over previous
import functools

import jax
import jax.numpy as jnp
from jax import lax
from jax.experimental import pallas as pl
from jax.experimental.pallas import tpu as pltpu
from jax.experimental.pallas import tpu_sc as plsc

_NUM_CLASSES = 1000
_D = 128
_N = 65536

_NC = 2
_NS = 16
_NW = _NC * _NS
_ROWS_W = _N // _NW
_CHUNK = 128
_NCHUNK = _ROWS_W // _CHUNK
_CPAD = 1024
_STRIPE = _CPAD // _NS


def _sc_stats(x, y, z_d, ones_rows):
  mesh = plsc.VectorSubcoreMesh(core_axis_name="c", subcore_axis_name="s")

  @functools.partial(
      pl.kernel,
      out_type=(
          jax.ShapeDtypeStruct((_NC, _CPAD, _D), jnp.float32),
          jax.ShapeDtypeStruct((_NC, _CPAD, _D), jnp.float32),
          jax.ShapeDtypeStruct((_NC, _CPAD, _D), jnp.float32),
      ),
      mesh=mesh,
      scratch_types=[
          pltpu.VMEM((_CHUNK, _D), jnp.float32),
          pltpu.VMEM((_CHUNK, _D), jnp.float32),
          pltpu.VMEM((_CHUNK, _D), jnp.float32),
          pltpu.VMEM((_CHUNK,), jnp.int32),
          pltpu.VMEM_SHARED((_CPAD, _D), jnp.float32),
          pltpu.VMEM_SHARED((_CPAD, _D), jnp.float32),
          pltpu.VMEM_SHARED((_CPAD, _D), jnp.float32),
      ],
  )
  def stats_kernel(x_hbm, y_hbm, zd_hbm, ones_hbm, s_out, t_out, c_out,
                   xb, qb, ob, ib, s_tab, t_tab, c_tab):
    cid = lax.axis_index("c")
    sid = lax.axis_index("s")
    wid = sid * _NC + cid
    base = wid * _ROWS_W
    r0 = sid * _STRIPE

    pltpu.sync_copy(zd_hbm.at[pl.ds(r0, _STRIPE)], s_tab.at[pl.ds(r0, _STRIPE)])
    pltpu.sync_copy(zd_hbm.at[pl.ds(r0, _STRIPE)], t_tab.at[pl.ds(r0, _STRIPE)])
    pltpu.sync_copy(zd_hbm.at[pl.ds(r0, _STRIPE)], c_tab.at[pl.ds(r0, _STRIPE)])

    pltpu.sync_copy(ones_hbm, ob)

    plsc.subcore_barrier()

    def chunk_body(k, carry):
      off = base + k * _CHUNK
      pltpu.sync_copy(y_hbm.at[pl.ds(off, _CHUNK)], ib)
      pltpu.sync_copy(x_hbm.at[pl.ds(off, _CHUNK)], xb)

      def sq_row(i, c2):
        for j in range(_D // 16):
          v = xb[i, pl.ds(j * 16, 16)]
          qb[i, pl.ds(j * 16, 16)] = v * v
        return c2

      lax.fori_loop(0, _CHUNK, sq_row, 0)

      pltpu.sync_copy(xb, s_tab.at[ib], add=True)
      pltpu.sync_copy(qb, t_tab.at[ib], add=True)
      pltpu.sync_copy(ob, c_tab.at[ib], add=True)
      return carry

    lax.fori_loop(0, _NCHUNK, chunk_body, 0)

    plsc.subcore_barrier()

    pltpu.sync_copy(s_tab.at[pl.ds(r0, _STRIPE)], s_out.at[cid, pl.ds(r0, _STRIPE)])
    pltpu.sync_copy(t_tab.at[pl.ds(r0, _STRIPE)], t_out.at[cid, pl.ds(r0, _STRIPE)])
    pltpu.sync_copy(c_tab.at[pl.ds(r0, _STRIPE)], c_out.at[cid, pl.ds(r0, _STRIPE)])

  return stats_kernel(x, y, z_d, ones_rows)


def _finalize(s2, t2, c2, anchors):
  def fin_kernel(s_ref, t_ref, c_ref, a_ref, o_ref):
    s = s_ref[0] + s_ref[1]
    t = t_ref[0] + t_ref[1]
    c = c_ref[0] + c_ref[1]
    a = a_ref[...]
    sv = s[:_NUM_CLASSES]
    qv = jnp.sum(t[:_NUM_CLASSES], axis=1, keepdims=True)
    cnt = c[:_NUM_CLASSES, 0:1]
    dot = jnp.sum(sv * a, axis=1, keepdims=True)
    na = jnp.sum(a * a, axis=1, keepdims=True)
    num = qv - 2.0 * dot + cnt * na
    w = jnp.where(cnt > 0.0, 1.0 / (cnt * cnt), 0.0)
    o_ref[0, 0] = jnp.sum(num * w)

  out = pl.pallas_call(
      fin_kernel,
      out_shape=jax.ShapeDtypeStruct((1, 1), jnp.float32),
      out_specs=pl.BlockSpec(memory_space=pltpu.SMEM),
  )(s2, t2, c2, anchors)
  return out[0, 0]


@jax.jit
def kernel(x, y, anchors):
  y32 = y.astype(jnp.int32)
  z_d = jnp.zeros((_CPAD, _D), jnp.float32)
  ones_rows = jnp.zeros((_CHUNK, _D), jnp.float32).at[:, 0].set(1.0)
  s2, t2, c2 = _sc_stats(x, y32, z_d, ones_rows)
  return _finalize(s2, t2, c2, anchors)


# speedup vs baseline: 9.4411x; 1.4517x over previous
import functools

import jax
import jax.numpy as jnp
from jax import lax
from jax.experimental import pallas as pl
from jax.experimental.pallas import tpu as pltpu
from jax.experimental.pallas import tpu_sc as plsc

_NUM_CLASSES = 1000
_D = 128
_N = 65536

_NC = 2
_NS = 16
_NW = _NC * _NS
_ROWS_W = _N // _NW
_CHUNK = 128
_NCHUNK = _ROWS_W // _CHUNK
_CPAD = 1024
_STRIPE = _CPAD // _NS
_CGRID = _CPAD // _D


def _sc_stats(x, y, z_d, idx8):
  mesh = plsc.VectorSubcoreMesh(core_axis_name="c", subcore_axis_name="s")

  @functools.partial(
      pl.kernel,
      out_type=(
          jax.ShapeDtypeStruct((_NC, _CPAD, _D), jnp.float32),
          jax.ShapeDtypeStruct((_NC, _CPAD, _D), jnp.float32),
          jax.ShapeDtypeStruct((_NC, _CGRID, _D), jnp.float32),
      ),
      mesh=mesh,
      compiler_params=pltpu.CompilerParams(needs_layout_passes=False),
      scratch_types=[
          pltpu.VMEM((2, _CHUNK, _D), jnp.float32),
          pltpu.VMEM((2, _CHUNK, _D), jnp.float32),
          pltpu.VMEM((2, _CHUNK), jnp.int32),
          pltpu.VMEM((_CPAD,), jnp.float32),
          pltpu.VMEM((_CGRID, _D), jnp.float32),
          pltpu.VMEM((_CGRID,), jnp.int32),
          pltpu.VMEM_SHARED((_CPAD, _D), jnp.float32),
          pltpu.VMEM_SHARED((_CPAD, _D), jnp.float32),
          pltpu.VMEM_SHARED((_CGRID, _D), jnp.float32),
          pltpu.SemaphoreType.DMA,
          pltpu.SemaphoreType.DMA,
          pltpu.SemaphoreType.DMA,
          pltpu.SemaphoreType.DMA,
          pltpu.SemaphoreType.DMA,
          pltpu.SemaphoreType.DMA,
          pltpu.SemaphoreType.DMA,
          pltpu.SemaphoreType.DMA,
      ],
  )
  def stats_kernel(x_hbm, y_hbm, zd_hbm, idx8_hbm, s_out, t_out, c_out,
                   xb, qb, ib, cnt, cgrid, i8, s_tab, t_tab, c_grid,
                   sx0, sx1, sy0, sy1, ss0, ss1, st0, st1):
    cid = lax.axis_index("c")
    sid = lax.axis_index("s")
    wid = sid * _NC + cid
    base = wid * _ROWS_W
    r0 = sid * _STRIPE
    sx = (sx0, sx1)
    sy = (sy0, sy1)
    ss = (ss0, ss1)
    st = (st0, st1)

    pltpu.sync_copy(zd_hbm.at[pl.ds(r0, _STRIPE)], s_tab.at[pl.ds(r0, _STRIPE)])
    pltpu.sync_copy(zd_hbm.at[pl.ds(r0, _STRIPE)], t_tab.at[pl.ds(r0, _STRIPE)])

    @pl.when(sid == 0)
    def _():
      pltpu.sync_copy(zd_hbm.at[pl.ds(0, _CGRID)], c_grid)

    pltpu.sync_copy(idx8_hbm, i8)

    zero16 = jnp.zeros((16,), jnp.float32)
    for j in range(_CPAD // 16):
      cnt[pl.ds(j * 16, 16)] = zero16

    plsc.subcore_barrier()

    ones16 = jnp.full((16,), 1.0, jnp.float32)

    def issue_in(k):
      b = k % 2
      off = base + k * _CHUNK
      dx = pltpu.async_copy(x_hbm.at[pl.ds(off, _CHUNK)], xb.at[b], sx[b])
      dy = pltpu.async_copy(y_hbm.at[pl.ds(off, _CHUNK)], ib.at[b], sy[b])
      return dx, dy

    in_descs = [None] * _NCHUNK
    out_descs = [None] * _NCHUNK
    in_descs[0] = issue_in(0)

    for k in range(_NCHUNK):
      b = k % 2
      dx, dy = in_descs[k]
      dx.wait()
      dy.wait()

      for g in range(_CHUNK // 16):
        yv = ib[b, pl.ds(g * 16, 16)]
        plsc.addupdate_scatter(cnt, [yv], ones16)

      def sq_row(i, c2, b=b):
        for j in range(_D // 16):
          v = xb[b, i, pl.ds(j * 16, 16)]
          qb[b, i, pl.ds(j * 16, 16)] = v * v
        return c2

      lax.fori_loop(0, _CHUNK, sq_row, 0)

      if k >= 1:
        ds_, dt_ = out_descs[k - 1]
        ds_.wait()
        dt_.wait()
      if k + 1 < _NCHUNK:
        in_descs[k + 1] = issue_in(k + 1)

      d_s = pltpu.async_copy(xb.at[b], s_tab.at[ib.at[b]], ss[b], add=True)
      d_t = pltpu.async_copy(qb.at[b], t_tab.at[ib.at[b]], st[b], add=True)
      out_descs[k] = (d_s, d_t)

    ds_, dt_ = out_descs[_NCHUNK - 1]
    ds_.wait()
    dt_.wait()

    for r in range(_CGRID):
      for j in range(_D // 16):
        cgrid[r, pl.ds(j * 16, 16)] = cnt[pl.ds(r * _D + j * 16, 16)]
    pltpu.sync_copy(cgrid, c_grid.at[i8], add=True)

    plsc.subcore_barrier()

    pltpu.sync_copy(s_tab.at[pl.ds(r0, _STRIPE)], s_out.at[cid, pl.ds(r0, _STRIPE)])
    pltpu.sync_copy(t_tab.at[pl.ds(r0, _STRIPE)], t_out.at[cid, pl.ds(r0, _STRIPE)])

    @pl.when(sid == 0)
    def _():
      pltpu.sync_copy(c_grid, c_out.at[cid])

  return stats_kernel(x, y, z_d, idx8)


def _finalize(s2, t2, c2, anchors):
  def fin_kernel(s_ref, t_ref, c_ref, a_ref, o_ref):
    s = s_ref[0] + s_ref[1]
    t = t_ref[0] + t_ref[1]
    cnt = c_ref[0, :_NUM_CLASSES] + c_ref[1, :_NUM_CLASSES]
    a = a_ref[...]
    sv = s[:_NUM_CLASSES]
    qv = jnp.sum(t[:_NUM_CLASSES], axis=1, keepdims=True)
    dot = jnp.sum(sv * a, axis=1, keepdims=True)
    na = jnp.sum(a * a, axis=1, keepdims=True)
    num = qv - 2.0 * dot + cnt * na
    w = jnp.where(cnt > 0.0, 1.0 / (cnt * cnt), 0.0)
    o_ref[0, 0] = jnp.sum(num * w)

  out = pl.pallas_call(
      fin_kernel,
      out_shape=jax.ShapeDtypeStruct((1, 1), jnp.float32),
      out_specs=pl.BlockSpec(memory_space=pltpu.SMEM),
  )(s2, t2, c2, anchors)
  return out[0, 0]


@jax.jit
def kernel(x, y, anchors):
  y32 = y.astype(jnp.int32)
  z_d = jnp.zeros((_CPAD, _D), jnp.float32)
  idx8 = jnp.arange(_CGRID, dtype=jnp.int32)
  s2, t2, c2 = _sc_stats(x, y32, z_d, idx8)
  c2col = c2.reshape(_NC, _CPAD, 1)
  return _finalize(s2, t2, c2col, anchors)
